# baseline (speedup 1.0000x reference)
import functools

import jax
import jax.numpy as jnp
from jax import lax
from jax.experimental import pallas as pl
from jax.experimental.pallas import tpu as pltpu
from jax.experimental.pallas import tpu_sc as plsc

VOCAB = 1000000
EMB = 64
BATCH = 16384
CTX = 20

NC = 2
NS = 16
NW = NC * NS
HALF_V = VOCAB // 2

BPW = BATCH // NW
RC = 16
NCH = BPW // RC
NQ = RC // 16
OBW = 128
INV_CTX = 1.0 / CTX


def _make_mesh():
    return plsc.VectorSubcoreMesh(
        core_axis_name="c", subcore_axis_name="s", num_cores=NC, num_subcores=NS
    )


_scratch = [
    pltpu.VMEM((BPW * CTX,), jnp.int32),
    pltpu.VMEM((CTX * RC, 2 * EMB), jnp.float32),
    pltpu.VMEM((EMB, OBW), jnp.float32),
    pltpu.SemaphoreType.DMA,
]


def _cbow_body(x_hbm, pk_hbm, out_hbm, idx_flat, stg, obuf, gsem):
    wid = lax.axis_index("s") * NC + lax.axis_index("c")
    base = wid * BPW

    pltpu.sync_copy(x_hbm.at[pl.ds(base * CTX, BPW * CTX)], idx_flat)

    iota16 = lax.iota(jnp.int32, 16)
    step = iota16 * CTX

    @pl.loop(0, NCH)
    def _chunk(cc):
        lanes0 = step + cc * (RC * CTX)
        for g in range(CTX):
            v = plsc.load_gather(idx_flat, [lanes0 + g])
            pltpu.async_copy(
                pk_hbm.at[lax.shift_right_logical(v, 1)],
                stg.at[pl.ds(g * RC, RC)],
                gsem,
            )
        for g in range(CTX):
            pltpu.make_async_copy(
                pk_hbm.at[iota16],
                stg.at[pl.ds(g * RC, RC)],
                gsem,
            ).wait()

        cols = [
            (plsc.load_gather(idx_flat, [lanes0 + g]) & 1) * EMB
            for g in range(CTX)
        ]
        rows = [iota16 + g * RC for g in range(CTX)]
        ob_col = (cc % 8) * RC

        @pl.loop(0, EMB)
        def _acc(e):
            s = plsc.load_gather(stg, [rows[0], cols[0] + e])
            for g in range(1, CTX):
                s = s + plsc.load_gather(stg, [rows[g], cols[g] + e])
            obuf[e, pl.ds(ob_col, 16)] = s * INV_CTX

        @pl.when((cc % 8) == 7)
        def _flush():
            pltpu.sync_copy(
                obuf,
                out_hbm.at[pl.ds(0, EMB), pl.ds(base + (cc // 8) * OBW, OBW)],
            )


_cache = []


def _get_cbow_sc():
    if not _cache:
        _cache.append(
            pl.kernel(
                _cbow_body,
                mesh=_make_mesh(),
                out_type=jax.ShapeDtypeStruct((EMB, BATCH), jnp.float32),
                scratch_types=_scratch,
                compiler_params=pltpu.CompilerParams(
                    needs_layout_passes=False, use_tc_tiling_on_sc=True
                ),
            )
        )
    return _cache[0]


def kernel(x, embedding_table):
    packed = embedding_table.reshape(HALF_V, 2 * EMB)
    out_t = _get_cbow_sc()(x.reshape(BATCH * CTX), packed)
    return out_t.T


# speedup vs baseline: 1.6218x; 1.6218x over previous
import functools

import jax
import jax.numpy as jnp
from jax import lax
from jax.experimental import pallas as pl
from jax.experimental.pallas import tpu as pltpu
from jax.experimental.pallas import tpu_sc as plsc

VOCAB = 1000000
EMB = 64
BATCH = 16384
CTX = 20

NC = 2
NS = 16
NW = NC * NS
BPW = BATCH // NW
CHUNK = 128
NJ = BPW // CHUNK
INV_CTX = 1.0 / CTX


def _make_mesh():
    return plsc.VectorSubcoreMesh(
        core_axis_name="c", subcore_axis_name="s", num_cores=NC, num_subcores=NS
    )


_scratch = [
    pltpu.VMEM((BPW * CTX,), jnp.int32),
    pltpu.VMEM((CTX * NJ, CHUNK), jnp.int32),
    pltpu.VMEM((BPW, EMB), jnp.float32),
    pltpu.SemaphoreType.DMA,
    pltpu.SemaphoreType.DMA,
]


def _cbow_body(x_hbm, table_hbm, out_hbm, idx_flat, idx_t, acc, isem, gsem):
    wid = lax.axis_index("s") * NC + lax.axis_index("c")
    base = wid * BPW

    idx_cp = pltpu.async_copy(
        x_hbm.at[pl.ds(base * CTX, BPW * CTX)], idx_flat, isem
    )

    zeros = jnp.zeros((16,), jnp.float32)

    @pl.loop(0, BPW)
    def _zero(r):
        for v in range(EMB // 16):
            acc[r, pl.ds(v * 16, 16)] = zeros

    idx_cp.wait()

    iota16 = lax.iota(jnp.int32, 16)
    step = iota16 * CTX

    @pl.loop(0, CTX)
    def _launch(g):
        for j in range(NJ):
            for t in range(CHUNK // 16):
                lanes = step + ((j * CHUNK + t * 16) * CTX + g)
                idx_t[g * NJ + j, pl.ds(t * 16, 16)] = plsc.load_gather(
                    idx_flat, [lanes]
                )
        for j in range(NJ):
            pltpu.async_copy(
                table_hbm.at[idx_t.at[g * NJ + j]],
                acc.at[pl.ds(j * CHUNK, CHUNK)],
                gsem,
                add=True,
            )

    @pl.loop(0, CTX * NJ)
    def _drain(i):
        pltpu.make_async_copy(
            table_hbm.at[idx_t.at[0]], acc.at[pl.ds(0, CHUNK)], gsem
        ).wait()

    @pl.loop(0, BPW)
    def _scale(r):
        for v in range(EMB // 16):
            sl = pl.ds(v * 16, 16)
            acc[r, sl] = acc[r, sl] * INV_CTX

    pltpu.sync_copy(acc, out_hbm.at[pl.ds(base, BPW)])


_cbow_sc_cache = []


def _get_cbow_sc():
    if not _cbow_sc_cache:
        _cbow_sc_cache.append(
            pl.kernel(
                _cbow_body,
                mesh=_make_mesh(),
                out_type=jax.ShapeDtypeStruct((BATCH, EMB), jnp.float32),
                scratch_types=_scratch,
                compiler_params=pltpu.CompilerParams(
                    needs_layout_passes=False, use_tc_tiling_on_sc=False
                ),
            )
        )
    return _cbow_sc_cache[0]


def kernel(x, embedding_table):
    return _get_cbow_sc()(x.reshape(BATCH * CTX), embedding_table)
